# baseline (speedup 1.0000x reference)
import functools

import jax
import jax.numpy as jnp
from jax import lax
from jax.experimental import pallas as pl
from jax.experimental.pallas import tpu as pltpu
from jax.experimental.pallas import tpu_sc as plsc

N = 10000
E = 320000
DIN = 128
H = 64
G = 64

NC = 2
NS = 16
CW = 128
EPAD = NS * 160 * CW
NPAD = N + 8
RB = N // 10
NBUF = 4
SP = 40
NRS = N // NS


def _r(a):
    return a.astype(jnp.bfloat16).astype(jnp.float32)


def _dotd(a, b):
    return jax.lax.dot_general(_r(a), _r(b), (((1,), (0,)), ((), ())),
                               preferred_element_type=jnp.float32)


def _dote(a, b):
    return jax.lax.dot_general(a, b, (((1,), (0,)), ((), ())),
                               preferred_element_type=jnp.float32)


_SC_PARAMS = pltpu.CompilerParams(use_tc_tiling_on_sc=False)
_MESH = plsc.VectorSubcoreMesh(core_axis_name="c", subcore_axis_name="s")


def _edge_ring(y_s, agg_s, src_hbm, dst_hbm, src_v, dst_v, rows, gsem, ssem,
               row0, nstage, cnt=None):
    def g_cp(j, b):
        return pltpu.make_async_copy(y_s.at[src_v.at[j]], rows[b], gsem[b])

    def s_cp(j, b):
        return pltpu.make_async_copy(rows[b], agg_s.at[dst_v.at[j]], ssem[b])

    def cnt_fire(j):
        if cnt is None:
            return
        cond, cnt_s, ones_v, csem = cnt

        @pl.when(cond)
        def _():
            pltpu.async_copy(ones_v.at[pl.ds(0, CW)], cnt_s.at[dst_v.at[j]],
                             csem, add=True)

    def cnt_drain(k):
        if cnt is None:
            return
        cond, cnt_s, ones_v, csem = cnt

        @pl.when(cond)
        def _():
            @pl.loop(0, k)
            def _(_i):
                pltpu.make_async_copy(ones_v.at[pl.ds(0, CW)],
                                      cnt_s.at[dst_v.at[0]], csem).wait()

    for st in range(nstage):
        off = row0 + st * SP
        pltpu.sync_copy(src_hbm.at[pl.ds(off, SP)], src_v)
        pltpu.sync_copy(dst_hbm.at[pl.ds(off, SP)], dst_v)

        for b in range(NBUF):
            pltpu.async_copy(y_s.at[src_v.at[b]], rows[b], gsem[b])

        @pl.loop(0, SP - NBUF, step=NBUF)
        def _(j):
            for b in range(NBUF):
                g_cp(j + b, b).wait()
                pltpu.async_copy(rows[b], agg_s.at[dst_v.at[j + b]],
                                 ssem[b], add=True)
                cnt_fire(j + b)
            for b in range(NBUF):
                s_cp(j + b, b).wait()
                pltpu.async_copy(y_s.at[src_v.at[j + b + NBUF]], rows[b],
                                 gsem[b])
            cnt_drain(NBUF)

        for b in range(NBUF):
            g_cp(SP - NBUF + b, b).wait()
            pltpu.async_copy(rows[b], agg_s.at[dst_v.at[SP - NBUF + b]],
                             ssem[b], add=True)
            cnt_fire(SP - NBUF + b)
        for b in range(NBUF):
            s_cp(SP - NBUF + b, b).wait()
        cnt_drain(NBUF)


def _fill(ref, n, val):
    @pl.loop(0, n, step=16)
    def _(i):
        ref[pl.ds(i, 16)] = jnp.full((16,), val, jnp.float32)


@functools.partial(
    pl.kernel,
    out_type=[jax.ShapeDtypeStruct((N, H), jnp.float32),
              jax.ShapeDtypeStruct((N, H), jnp.float32),
              jax.ShapeDtypeStruct((N // RB, 1, RB), jnp.float32)],
    mesh=_MESH,
    compiler_params=_SC_PARAMS,
    scratch_types=[
        pltpu.VMEM_SHARED((N, H), jnp.float32),
        pltpu.VMEM_SHARED((NPAD, H), jnp.float32),
        pltpu.VMEM((SP, CW), jnp.int32),
        pltpu.VMEM((SP, CW), jnp.int32),
        [pltpu.VMEM((CW, H), jnp.float32)] * NBUF,
        [pltpu.SemaphoreType.DMA] * NBUF,
        [pltpu.SemaphoreType.DMA] * NBUF,
        pltpu.VMEM_SHARED((NPAD,), jnp.float32),
        pltpu.VMEM((128,), jnp.float32),
        pltpu.VMEM((RB + 8,), jnp.float32),
        pltpu.SemaphoreType.DMA,
    ])
def _sc_agg_x(x0_hbm, x1_hbm, src_hbm, dst_hbm, za_hbm,
              a0_hbm, a1_hbm, c_hbm,
              y_s, agg_s, src_v, dst_v, rows, gsem, ssem,
              cnt_s, ones_v, zb_v, csem):
    cid = lax.axis_index("c")
    sid = lax.axis_index("s")
    rr0 = sid * NRS
    r0 = sid * RB
    is0 = cid == 0

    _fill(ones_v, 128, 1.0)
    _fill(zb_v, RB + 8, 0.0)

    @pl.when(is0)
    def _():
        pltpu.sync_copy(x0_hbm.at[pl.ds(rr0, NRS)], y_s.at[pl.ds(rr0, NRS)])

        @pl.when(sid < 10)
        def _():
            pltpu.sync_copy(zb_v.at[pl.ds(0, RB)], cnt_s.at[pl.ds(r0, RB)])

    @pl.when(cid == 1)
    def _():
        pltpu.sync_copy(x1_hbm.at[pl.ds(rr0, NRS)], y_s.at[pl.ds(rr0, NRS)])

    pltpu.sync_copy(za_hbm.at[pl.ds(rr0, NRS)], agg_s.at[pl.ds(rr0, NRS)])
    plsc.subcore_barrier()

    _edge_ring(y_s, agg_s, src_hbm, dst_hbm, src_v, dst_v, rows, gsem, ssem,
               row0=sid * 160, nstage=4, cnt=(is0, cnt_s, ones_v, csem))

    plsc.subcore_barrier()

    @pl.when(is0)
    def _():
        pltpu.sync_copy(agg_s.at[pl.ds(rr0, NRS)], a0_hbm.at[pl.ds(rr0, NRS)])

        @pl.when(sid < 10)
        def _():
            pltpu.sync_copy(cnt_s.at[pl.ds(r0, RB)], zb_v.at[pl.ds(0, RB)])
            pltpu.sync_copy(zb_v.at[pl.ds(0, RB)], c_hbm.at[sid, 0])

    @pl.when(cid == 1)
    def _():
        pltpu.sync_copy(agg_s.at[pl.ds(rr0, NRS)], a1_hbm.at[pl.ds(rr0, NRS)])


@functools.partial(
    pl.kernel,
    out_type=[jax.ShapeDtypeStruct((N, H), jnp.float32),
              jax.ShapeDtypeStruct((N, H), jnp.float32)],
    mesh=_MESH,
    compiler_params=_SC_PARAMS,
    scratch_types=[
        pltpu.VMEM_SHARED((N, H), jnp.float32),
        pltpu.VMEM_SHARED((NPAD, H), jnp.float32),
        pltpu.VMEM((SP, CW), jnp.int32),
        pltpu.VMEM((SP, CW), jnp.int32),
        [pltpu.VMEM((CW, H), jnp.float32)] * NBUF,
        [pltpu.SemaphoreType.DMA] * NBUF,
        [pltpu.SemaphoreType.DMA] * NBUF,
    ])
def _sc_agg_h(h_hbm, src_hbm, dst_hbm, za_hbm, p0_hbm, p1_hbm,
              y_s, agg_s, src_v, dst_v, rows, gsem, ssem):
    cid = lax.axis_index("c")
    sid = lax.axis_index("s")
    rr0 = sid * NRS

    pltpu.sync_copy(h_hbm.at[pl.ds(rr0, NRS)], y_s.at[pl.ds(rr0, NRS)])
    pltpu.sync_copy(za_hbm.at[pl.ds(rr0, NRS)], agg_s.at[pl.ds(rr0, NRS)])
    plsc.subcore_barrier()

    _edge_ring(y_s, agg_s, src_hbm, dst_hbm, src_v, dst_v, rows, gsem, ssem,
               row0=(cid * NS + sid) * 80, nstage=2)

    plsc.subcore_barrier()

    @pl.when(cid == 0)
    def _():
        pltpu.sync_copy(agg_s.at[pl.ds(rr0, NRS)], p0_hbm.at[pl.ds(rr0, NRS)])

    @pl.when(cid == 1)
    def _():
        pltpu.sync_copy(agg_s.at[pl.ds(rr0, NRS)], p1_hbm.at[pl.ds(rr0, NRS)])


_NB = 10
_BN = N // _NB


def _tc_h1(a0, a1, c3, x, b1, wl1t, wl1b, wr1):
    def body(a0r, a1r, cr, xr, b1r, wtr, wbr, wrr, hr):
        cnt = jnp.maximum(cr[0, 0], 1.0)[:, None]
        d = _dotd(a0r[...] / cnt, wtr[...]) + _dotd(a1r[...] / cnt, wbr[...])
        hr[...] = jnp.maximum(d + b1r[...] + _dotd(xr[...], wrr[...]), 0.0)

    return pl.pallas_call(
        body,
        grid=(_NB,),
        in_specs=[pl.BlockSpec((_BN, H), lambda i: (i, 0)),
                  pl.BlockSpec((_BN, H), lambda i: (i, 0)),
                  pl.BlockSpec((1, 1, _BN), lambda i: (i, 0, 0)),
                  pl.BlockSpec((_BN, DIN), lambda i: (i, 0)),
                  pl.BlockSpec((1, H), lambda i: (0, 0)),
                  pl.BlockSpec((H, H), lambda i: (0, 0)),
                  pl.BlockSpec((H, H), lambda i: (0, 0)),
                  pl.BlockSpec((DIN, H), lambda i: (0, 0))],
        out_specs=pl.BlockSpec((_BN, H), lambda i: (i, 0)),
        out_shape=jax.ShapeDtypeStruct((N, H), jnp.float32),
    )(a0, a1, c3, x, b1, wl1t, wl1b, wr1)


def _tc_final(q0, q1, c3, h, b2, wl2, wr2, batch3, wrow, brow):
    def body(q0r, q1r, cr, hr, b2r, wlr, wrr, br, wror, bror, out_ref, acc):
        i = pl.program_id(0)

        @pl.when(i == 0)
        def _():
            acc[...] = jnp.zeros_like(acc)

        cnt = jnp.maximum(cr[0, 0], 1.0)[:, None]
        h2 = jnp.maximum(_dotd((q0r[...] + q1r[...]) / cnt, wlr[...])
                         + b2r[...] + _dotd(hr[...], wrr[...]), 0.0)
        h2_hi = _r(h2)
        h2_lo = h2 - h2_hi
        hcat = jnp.concatenate(
            [h2_hi, jnp.ones((_BN, 1), jnp.float32),
             jnp.zeros((_BN, DIN - H - 1), jnp.float32)], axis=1)
        lcat = jnp.concatenate(
            [h2_lo, jnp.zeros((_BN, DIN - H), jnp.float32)], axis=1)
        b = br[0, 0]
        mask = (lax.broadcasted_iota(jnp.int32, (G, _BN), 0)
                == b[None, :]).astype(jnp.float32)
        acc[...] += _dote(mask, hcat) + _dote(mask, lcat)

        @pl.when(i == _NB - 1)
        def _():
            pooled = acc[:, :H] / jnp.maximum(acc[:, H:H + 1], 1.0)
            out_ref[...] = (jnp.sum(_r(pooled) * _r(wror[...]), axis=1)
                            + bror[0])

    return pl.pallas_call(
        body,
        grid=(_NB,),
        in_specs=[pl.BlockSpec((_BN, H), lambda i: (i, 0)),
                  pl.BlockSpec((_BN, H), lambda i: (i, 0)),
                  pl.BlockSpec((1, 1, _BN), lambda i: (i, 0, 0)),
                  pl.BlockSpec((_BN, H), lambda i: (i, 0)),
                  pl.BlockSpec((1, H), lambda i: (0, 0)),
                  pl.BlockSpec((H, H), lambda i: (0, 0)),
                  pl.BlockSpec((H, H), lambda i: (0, 0)),
                  pl.BlockSpec((1, 1, _BN), lambda i: (i, 0, 0)),
                  pl.BlockSpec((1, H), lambda i: (0, 0)),
                  pl.BlockSpec((1, H), lambda i: (0, 0))],
        out_specs=pl.BlockSpec((G,), lambda i: (0,)),
        out_shape=jax.ShapeDtypeStruct((G,), jnp.float32),
        scratch_shapes=[pltpu.VMEM((G, DIN), jnp.float32)],
    )(q0, q1, c3, h, b2, wl2, wr2, batch3, wrow, brow)


def kernel(x, edge_index, batch, W_l1, b_l1, W_r1, W_l2, b_l2, W_r2,
           W_out, b_out):
    npad = EPAD - E
    src2 = jnp.concatenate(
        [edge_index[0], jnp.zeros((npad,), jnp.int32)]).reshape(EPAD // CW, CW)
    dst2 = jnp.concatenate(
        [edge_index[1], jnp.full((npad,), N, jnp.int32)]).reshape(EPAD // CW,
                                                                  CW)
    za = jnp.zeros((N, H), jnp.float32)
    x0 = x[:, :H]
    x1 = x[:, H:]
    wl1t = W_l1[:H]
    wl1b = W_l1[H:]
    b1 = b_l1.reshape(1, H)
    b2 = b_l2.reshape(1, H)
    batch3 = batch.reshape(_NB, 1, _BN)
    wrow = W_out.reshape(1, H)
    brow = jnp.broadcast_to(b_out.reshape(1, 1), (1, H))

    a0, a1, c3 = _sc_agg_x(x0, x1, src2, dst2, za)
    h = _tc_h1(a0, a1, c3, x, b1, wl1t, wl1b, wr1=W_r1)
    q0, q1 = _sc_agg_h(h, src2, dst2, za)
    return _tc_final(q0, q1, c3, h, b2, W_l2, W_r2, batch3, wrow, brow)
